# Initial kernel scaffold; baseline (speedup 1.0000x reference)
import jax
import jax.numpy as jnp
from jax.experimental import pallas as pl


def kernel(indices, table):
    raise NotImplementedError("write your pallas kernel here")


# speedup vs baseline: 1.0952x; 1.0952x over previous
import functools

import jax
import jax.numpy as jnp
from jax import lax
from jax.experimental import pallas as pl
from jax.experimental.pallas import tpu as pltpu
from jax.experimental.pallas import tpu_sc as plsc

HIDDEN = 32

NUM_CORES = 2
NUM_SUBCORES = 16
NW = NUM_CORES * NUM_SUBCORES

CHUNK = 128
GATHERS_PER_GRP = 8
GRP = CHUNK * GATHERS_PER_GRP


def _make_gather(vocab: int, total: int):
    b_per_w = total // NW
    n_grp = b_per_w // GRP
    mesh = plsc.VectorSubcoreMesh(core_axis_name="c", subcore_axis_name="s")

    @functools.partial(
        pl.kernel,
        mesh=mesh,
        out_type=jax.ShapeDtypeStruct((total, HIDDEN), jnp.float32),
        scratch_types=[
            pltpu.VMEM((GATHERS_PER_GRP, CHUNK), jnp.int32),
            pltpu.VMEM((GRP, HIDDEN), jnp.float32),
            pltpu.SemaphoreType.DMA,
        ],
        compiler_params=pltpu.CompilerParams(use_tc_tiling_on_sc=False),
    )
    def gather_kernel(idx_hbm, table_hbm, out_hbm, idx_v, rows_v, sem):
        wid = lax.axis_index("s") * NUM_CORES + lax.axis_index("c")
        chunk_base = wid * (b_per_w // CHUNK)
        row_base = wid * b_per_w

        def body(g, carry):
            pltpu.sync_copy(
                idx_hbm.at[pl.ds(chunk_base + g * GATHERS_PER_GRP,
                                 GATHERS_PER_GRP)],
                idx_v,
            )
            copies = []
            for j in range(GATHERS_PER_GRP):
                copies.append(
                    pltpu.async_copy(
                        table_hbm.at[idx_v.at[j]],
                        rows_v.at[pl.ds(j * CHUNK, CHUNK)],
                        sem,
                    )
                )
            for c in copies:
                c.wait()
            pltpu.sync_copy(
                rows_v,
                out_hbm.at[pl.ds(row_base + g * GRP, GRP)],
            )
            return carry

        lax.fori_loop(0, n_grp, body, 0)

    return gather_kernel


def kernel(indices, table):
    batch, hist = indices.shape
    total = batch * hist
    idx2d = indices.reshape(total // CHUNK, CHUNK).astype(jnp.int32)
    out = _make_gather(table.shape[0], total)(idx2d, table)
    return out.reshape(batch, hist, HIDDEN)


# speedup vs baseline: 1.1107x; 1.0141x over previous
import functools

import jax
import jax.numpy as jnp
from jax import lax
from jax.experimental import pallas as pl
from jax.experimental.pallas import tpu as pltpu
from jax.experimental.pallas import tpu_sc as plsc

HIDDEN = 32

NUM_CORES = 2
NUM_SUBCORES = 16
NW = NUM_CORES * NUM_SUBCORES

CHUNK = 128
G = 8
GRP = CHUNK * G


def _make_gather(total: int):
    b_per_w = total // NW
    n_grp = b_per_w // GRP
    n_pairs = (n_grp - 1) // 2
    chunks_per_w = b_per_w // CHUNK
    mesh = plsc.VectorSubcoreMesh(core_axis_name="c", subcore_axis_name="s")

    @functools.partial(
        pl.kernel,
        mesh=mesh,
        out_type=jax.ShapeDtypeStruct((total, HIDDEN), jnp.float32),
        scratch_types=[
            pltpu.VMEM((2, G, CHUNK), jnp.int32),
            pltpu.VMEM((2, GRP, HIDDEN), jnp.float32),
            pltpu.SemaphoreType.DMA,
            pltpu.SemaphoreType.DMA,
            pltpu.SemaphoreType.DMA,
            pltpu.SemaphoreType.DMA,
            pltpu.SemaphoreType.DMA,
        ],
        compiler_params=pltpu.CompilerParams(use_tc_tiling_on_sc=False),
    )
    def gather_kernel(idx_hbm, table_hbm, out_hbm, idx_v, rows_v,
                      sem_i0, sem_i1, sem_o0, sem_o1, sem_g):
        sem_idx = (sem_i0, sem_i1)
        sem_out = (sem_o0, sem_o1)
        wid = lax.axis_index("s") * NUM_CORES + lax.axis_index("c")
        chunk_base = wid * chunks_per_w
        row_base = wid * b_per_w

        def start_idx(g, b):
            pltpu.async_copy(
                idx_hbm.at[pl.ds(chunk_base + g * G, G)],
                idx_v.at[b], sem_idx[b],
            )

        def wait_idx(b):
            pltpu.make_async_copy(
                idx_hbm.at[pl.ds(chunk_base, G)],
                idx_v.at[b], sem_idx[b],
            ).wait()

        def run_gathers(b):
            copies = []
            for j in range(G):
                copies.append(
                    pltpu.async_copy(
                        table_hbm.at[idx_v.at[b].at[j]],
                        rows_v.at[b].at[pl.ds(j * CHUNK, CHUNK)],
                        sem_g,
                    )
                )
            for c in copies:
                c.wait()

        def start_out(g, b):
            pltpu.async_copy(
                rows_v.at[b],
                out_hbm.at[pl.ds(row_base + g * GRP, GRP)],
                sem_out[b],
            )

        def wait_out(b):
            pltpu.make_async_copy(
                rows_v.at[b],
                out_hbm.at[pl.ds(row_base, GRP)], sem_out[b],
            ).wait()

        start_idx(0, 0)
        start_idx(1, 1)
        wait_idx(0)
        run_gathers(0)
        start_idx(2, 0)
        start_out(0, 0)

        def pair_body(p, carry):
            for b in (1, 0):
                g = 2 * p + (1 if b == 1 else 2)
                wait_idx(b)

                @pl.when(g >= 2)
                def _wait_out():
                    wait_out(b)

                run_gathers(b)

                @pl.when(g + 2 < n_grp)
                def _prefetch_idx():
                    start_idx(g + 2, b)

                start_out(g, b)
            return carry

        lax.fori_loop(0, n_pairs, pair_body, 0)

        wait_out(0)
        wait_out(1)

    return gather_kernel


def kernel(indices, table):
    batch, hist = indices.shape
    total = batch * hist
    idx2d = indices.reshape(total // CHUNK, CHUNK).astype(jnp.int32)
    out = _make_gather(total)(idx2d, table)
    return out.reshape(batch, hist, HIDDEN)


# speedup vs baseline: 1.7984x; 1.6191x over previous
import functools

import jax
import jax.numpy as jnp
from jax import lax
from jax.experimental import pallas as pl
from jax.experimental.pallas import tpu as pltpu
from jax.experimental.pallas import tpu_sc as plsc

HIDDEN = 32

NUM_CORES = 2
NUM_SUBCORES = 16
NW = NUM_CORES * NUM_SUBCORES

GRP_B = 32


def _make_gather(batch: int, hist: int):
    b_per_w = batch // NW
    n_grp = b_per_w // GRP_B
    n_pairs = n_grp // 2
    mesh = plsc.VectorSubcoreMesh(core_axis_name="c", subcore_axis_name="s")

    @functools.partial(
        pl.kernel,
        mesh=mesh,
        out_type=jax.ShapeDtypeStruct((batch, hist, HIDDEN), jnp.float32),
        scratch_types=[
            pltpu.VMEM((2, GRP_B, hist), jnp.int32),
            pltpu.VMEM((2, GRP_B, hist, HIDDEN), jnp.float32),
            pltpu.SemaphoreType.DMA,
            pltpu.SemaphoreType.DMA,
            pltpu.SemaphoreType.DMA,
            pltpu.SemaphoreType.DMA,
            pltpu.SemaphoreType.DMA,
        ],
        compiler_params=pltpu.CompilerParams(use_tc_tiling_on_sc=False),
    )
    def gather_kernel(idx_hbm, table_hbm, out_hbm, idx_v, rows_v,
                      sem_i0, sem_i1, sem_o0, sem_o1, sem_g):
        sem_idx = (sem_i0, sem_i1)
        sem_out = (sem_o0, sem_o1)
        wid = lax.axis_index("s") * NUM_CORES + lax.axis_index("c")
        b_base = wid * b_per_w

        def start_idx(g, b):
            pltpu.async_copy(
                idx_hbm.at[pl.ds(b_base + g * GRP_B, GRP_B)],
                idx_v.at[b], sem_idx[b],
            )

        def wait_idx(b):
            pltpu.make_async_copy(
                idx_hbm.at[pl.ds(b_base, GRP_B)],
                idx_v.at[b], sem_idx[b],
            ).wait()

        def run_gathers(b):
            copies = []
            for j in range(GRP_B):
                copies.append(
                    pltpu.async_copy(
                        table_hbm.at[idx_v.at[b].at[j]],
                        rows_v.at[b].at[j],
                        sem_g,
                    )
                )
            for c in copies:
                c.wait()

        def start_out(g, b):
            pltpu.async_copy(
                rows_v.at[b],
                out_hbm.at[pl.ds(b_base + g * GRP_B, GRP_B)],
                sem_out[b],
            )

        def wait_out(b):
            pltpu.make_async_copy(
                rows_v.at[b],
                out_hbm.at[pl.ds(b_base, GRP_B)], sem_out[b],
            ).wait()

        start_idx(0, 0)
        start_idx(1, 1)

        def pair_body(p, carry):
            for b in range(2):
                g = 2 * p + b
                wait_idx(b)

                @pl.when(g >= 2)
                def _wait_out():
                    wait_out(b)

                run_gathers(b)

                @pl.when(g + 2 < n_grp)
                def _prefetch_idx():
                    start_idx(g + 2, b)

                start_out(g, b)
            return carry

        lax.fori_loop(0, n_pairs, pair_body, 0)

        wait_out(0)
        wait_out(1)

    return gather_kernel


def kernel(indices, table):
    batch, hist = indices.shape
    return _make_gather(batch, hist)(indices, table)
